# Initial kernel scaffold; baseline (speedup 1.0000x reference)
import jax
import jax.numpy as jnp
from jax.experimental import pallas as pl


def kernel(inputs):
    raise NotImplementedError("write your pallas kernel here")


# speedup vs baseline: 25.8511x; 25.8511x over previous
import jax
import jax.numpy as jnp
from jax.experimental import pallas as pl

_H = 32
_W = 32
_NITER = 2 * (_H + _W)
_PBLK = 128
_GBLK = 32


def _prop_kernel(x_ref, t_ref):
    x = x_ref[...]
    mask = (x != 0).astype(jnp.int32)
    hh = jax.lax.broadcasted_iota(jnp.int32, (_H, _W, 1), 0)
    ww = jax.lax.broadcasted_iota(jnp.int32, (_H, _W, 1), 1)
    t0 = (_H * _W) - hh * _W - ww
    t = mask * t0
    z_h = jnp.zeros((1, _W, _PBLK), jnp.int32)
    z_w = jnp.zeros((_H, 1, _PBLK), jnp.int32)

    def body(_, t):
        a = jnp.concatenate([t[1:], z_h], axis=0)
        b = jnp.concatenate([z_h, t[:-1]], axis=0)
        c = jnp.concatenate([t[:, 1:], z_w], axis=1)
        d = jnp.concatenate([z_w, t[:, :-1]], axis=1)
        return mask * jnp.maximum(
            jnp.maximum(jnp.maximum(a, b), jnp.maximum(c, d)), t)

    t_ref[...] = jax.lax.fori_loop(0, _NITER, body, t)


def _hist_kernel(t_ref, o_ref):
    t = t_ref[...]
    v = (_H * _W) - t
    hi = v >> 5
    lo = v & 31
    bg = jnp.sum((t == 0).astype(jnp.int32), axis=1, keepdims=True)
    iota_col = jax.lax.broadcasted_iota(jnp.int32, (_H, 1), 0)

    modes = []
    for g in range(_GBLK // 4):
        a_rows = []
        b_rows = []
        for s in range(4):
            p = 4 * g + s
            a_rows.append((hi[p:p + 1, :] == iota_col).astype(jnp.float32))
            b_rows.append((lo[p:p + 1, :] == iota_col).astype(jnp.float32))
        a4 = jnp.concatenate(a_rows, axis=0)
        b4 = jnp.concatenate(b_rows, axis=0)
        c4 = jax.lax.dot_general(
            a4, b4, (((1,), (1,)), ((), ())),
            preferred_element_type=jnp.float32)
        for s in range(4):
            p = 4 * g + s
            sub = c4[32 * s:32 * s + 32, 32 * s:32 * s + 32]
            cm = jnp.max(sub).astype(jnp.int32)
            modes.append(jnp.maximum(cm, bg[p, 0]))

    total = modes[0]
    for m in modes[1:]:
        total = total + m
    val = (total // _GBLK).astype(jnp.float32)
    o_ref[...] = jnp.broadcast_to(val, (1, 1, 128))


def kernel(inputs):
    n_cd, n_box, B, Pt, H, W = inputs.shape
    P = n_cd * n_box * B * Pt
    x = jnp.transpose(inputs.reshape(P, H, W), (1, 2, 0))

    t = pl.pallas_call(
        _prop_kernel,
        grid=(P // _PBLK,),
        in_specs=[pl.BlockSpec((_H, _W, _PBLK), lambda i: (0, 0, i))],
        out_specs=pl.BlockSpec((_H, _W, _PBLK), lambda i: (0, 0, i)),
        out_shape=jax.ShapeDtypeStruct((H, W, P), jnp.int32),
    )(x)

    tt = jnp.transpose(t, (2, 0, 1)).reshape(P, H * W)

    res = pl.pallas_call(
        _hist_kernel,
        grid=(P // _GBLK,),
        in_specs=[pl.BlockSpec((_GBLK, H * W), lambda i: (i, 0))],
        out_specs=pl.BlockSpec((1, 1, 128), lambda i: (i, 0, 0)),
        out_shape=jax.ShapeDtypeStruct((P // _GBLK, 1, 128), jnp.float32),
    )(tt)

    return res[:, 0, 0].reshape(n_cd, n_box, B)


# speedup vs baseline: 39.7321x; 1.5370x over previous
import jax
import jax.numpy as jnp
from jax.experimental import pallas as pl

_H = 32
_W = 32
_NITER = 2 * (_H + _W)
_CHUNK = 8
_PBLK = 128
_GBLK = 32


def _prop_kernel(x_ref, t_ref):
    x = x_ref[...]
    mask = (x != 0).astype(jnp.int32)
    hh = jax.lax.broadcasted_iota(jnp.int32, (_H, _W, 1), 0)
    ww = jax.lax.broadcasted_iota(jnp.int32, (_H, _W, 1), 1)
    t0 = (_H * _W) - hh * _W - ww
    t = mask * t0
    z_h = jnp.zeros((1, _W, _PBLK), jnp.int32)
    z_w = jnp.zeros((_H, 1, _PBLK), jnp.int32)

    def step(t):
        a = jnp.concatenate([t[1:], z_h], axis=0)
        b = jnp.concatenate([z_h, t[:-1]], axis=0)
        c = jnp.concatenate([t[:, 1:], z_w], axis=1)
        d = jnp.concatenate([z_w, t[:, :-1]], axis=1)
        return mask * jnp.maximum(
            jnp.maximum(jnp.maximum(a, b), jnp.maximum(c, d)), t)

    def w_cond(carry):
        _, it, _, changed = carry
        return jnp.logical_and(it < _NITER, changed)

    def w_body(carry):
        t, it, s_prev, _ = carry
        for _ in range(_CHUNK):
            t = step(t)
        s = jnp.sum(t)
        return t, it + _CHUNK, s, s != s_prev

    t, _, _, _ = jax.lax.while_loop(
        w_cond, w_body, (t, jnp.int32(0), jnp.sum(t), jnp.bool_(True)))
    t_ref[...] = t


def _hist_kernel(t_ref, o_ref):
    t = t_ref[...]
    v = (_H * _W) - t
    hi = v >> 5
    lo = v & 31
    bg = jnp.sum((t == 0).astype(jnp.int32), axis=1, keepdims=True)
    iota_col = jax.lax.broadcasted_iota(jnp.int32, (_H, 1), 0)

    modes = []
    for g in range(_GBLK // 4):
        a_rows = []
        b_rows = []
        for s in range(4):
            p = 4 * g + s
            a_rows.append((hi[p:p + 1, :] == iota_col).astype(jnp.float32))
            b_rows.append((lo[p:p + 1, :] == iota_col).astype(jnp.float32))
        a4 = jnp.concatenate(a_rows, axis=0)
        b4 = jnp.concatenate(b_rows, axis=0)
        c4 = jax.lax.dot_general(
            a4, b4, (((1,), (1,)), ((), ())),
            preferred_element_type=jnp.float32)
        for s in range(4):
            p = 4 * g + s
            sub = c4[32 * s:32 * s + 32, 32 * s:32 * s + 32]
            cm = jnp.max(sub).astype(jnp.int32)
            modes.append(jnp.maximum(cm, bg[p, 0]))

    total = modes[0]
    for m in modes[1:]:
        total = total + m
    val = (total // _GBLK).astype(jnp.float32)
    o_ref[...] = jnp.broadcast_to(val, (1, 1, 128))


def kernel(inputs):
    n_cd, n_box, B, Pt, H, W = inputs.shape
    P = n_cd * n_box * B * Pt
    x = jnp.transpose(inputs.reshape(P, H, W), (1, 2, 0))

    t = pl.pallas_call(
        _prop_kernel,
        grid=(P // _PBLK,),
        in_specs=[pl.BlockSpec((_H, _W, _PBLK), lambda i: (0, 0, i))],
        out_specs=pl.BlockSpec((_H, _W, _PBLK), lambda i: (0, 0, i)),
        out_shape=jax.ShapeDtypeStruct((H, W, P), jnp.int32),
    )(x)

    tt = jnp.transpose(t, (2, 0, 1)).reshape(P, H * W)

    res = pl.pallas_call(
        _hist_kernel,
        grid=(P // _GBLK,),
        in_specs=[pl.BlockSpec((_GBLK, H * W), lambda i: (i, 0))],
        out_specs=pl.BlockSpec((1, 1, 128), lambda i: (i, 0, 0)),
        out_shape=jax.ShapeDtypeStruct((P // _GBLK, 1, 128), jnp.float32),
    )(tt)

    return res[:, 0, 0].reshape(n_cd, n_box, B)
